# baseline (speedup 1.0000x reference)
import functools

import jax
import jax.numpy as jnp
from jax import lax
from jax.experimental import pallas as pl
from jax.experimental.pallas import tpu as pltpu
from jax.experimental.pallas import tpu_sc as plsc

B = 16384
D = 32
L = 16
NUM_CORES = 2
NUM_SUBCORES = 16
NW = NUM_CORES * NUM_SUBCORES
BPW = B // NW
CH = 4
NCH = BPW // CH
HALF = D // 2
NMT = 782
MPK = NMT * 32

_mesh = plsc.VectorSubcoreMesh(core_axis_name="c", subcore_axis_name="s")


@functools.partial(
    pl.kernel,
    mesh=_mesh,
    out_type=(
        jax.ShapeDtypeStruct((B,), jnp.float32),
        jax.ShapeDtypeStruct((NUM_CORES, MPK, 128), jnp.float32),
    ),
    scratch_types=[
        pltpu.VMEM((BPW + L,), jnp.int32),
        pltpu.VMEM((BPW + L,), jnp.int32),
        pltpu.VMEM((BPW,), jnp.int32),
        pltpu.VMEM((CH * D, 128), jnp.float32),
        pltpu.VMEM((D, 128), jnp.float32),
        pltpu.VMEM((D, 128), jnp.float32),
        pltpu.VMEM((BPW, 128), jnp.float32),
        pltpu.VMEM((HALF * BPW,), jnp.float32),
        pltpu.VMEM((BPW,), jnp.float32),
        pltpu.VMEM((BPW,), jnp.float32),
        pltpu.VMEM((BPW,), jnp.float32),
        pltpu.SemaphoreType.DMA,
    ],
    compiler_params=pltpu.CompilerParams(
        needs_layout_passes=False,
        disable_bounds_checks=True,
    ),
)
def _mf_sc(user_hbm, mission_hbm, uembT_hbm, membT_hbm, ubias_hbm, mbias_hbm,
           out_hbm, mscr_hbm, uidx_v, midx_v, mpk_v, ublk_v, inblk_v,
           outblk_v, mrows_v, prod_v, ub_v, mb_v, o_v, sem):
    cid = lax.axis_index("c")
    sid = lax.axis_index("s")
    wid = sid * NUM_CORES + cid
    base = wid * BPW

    pltpu.sync_copy(user_hbm.at[pl.ds(base, BPW)], uidx_v.at[pl.ds(0, BPW)])
    pltpu.sync_copy(mission_hbm.at[pl.ds(base, BPW)], midx_v.at[pl.ds(0, BPW)])
    uidx_v[pl.ds(BPW, L)] = jnp.zeros((L,), jnp.int32)
    midx_v[pl.ds(BPW, L)] = jnp.zeros((L,), jnp.int32)

    cp_ub = pltpu.async_copy(ubias_hbm.at[uidx_v.at[pl.ds(0, BPW)]], ub_v, sem)
    cp_mb = pltpu.async_copy(mbias_hbm.at[midx_v.at[pl.ds(0, BPW)]], mb_v, sem)

    iota = lax.iota(jnp.int32, L)

    def relayout_body(ti, carry):
        t = ti * NUM_SUBCORES + sid

        @pl.when(t < NMT)
        def _():
            off = pl.multiple_of(t * 128, 128)
            pltpu.sync_copy(membT_hbm.at[:, pl.ds(off, 128)], inblk_v)

            def q_body(q, carry2):
                for cg in range(8):
                    d_vec = iota + (cg & 1) * L
                    src_lane = jnp.broadcast_to(4 * q + cg // 2, (L,))
                    outblk_v[q, pl.ds(cg * L, L)] = plsc.load_gather(
                        inblk_v, [d_vec, src_lane])
                return carry2

            lax.fori_loop(0, D, q_body, 0)
            pltpu.sync_copy(outblk_v, mscr_hbm.at[cid, pl.ds(t * 32, 32), :])
        return carry

    lax.fori_loop(0, (NMT + NUM_SUBCORES - 1) // NUM_SUBCORES,
                  relayout_body, 0)

    def pack_body(i, carry):
        sl = pl.ds(i * L, L)
        mpk_v[sl] = lax.shift_right_logical(midx_v[sl], 2)
        return carry

    lax.fori_loop(0, BPW // L, pack_body, 0)
    plsc.subcore_barrier()

    cp_m = pltpu.async_copy(mscr_hbm.at[cid].at[mpk_v], mrows_v, sem)
    cp_ub.wait()
    cp_mb.wait()
    cp_m.wait()

    def chunk_body(c, carry):
        uvec = uidx_v[pl.ds(c * CH, L)]
        mvec = midx_v[pl.ds(c * CH, L)]
        for k in range(CH):
            ut = pl.multiple_of((uvec[k] >> 7) * 128, 128)
            pltpu.async_copy(uembT_hbm.at[:, pl.ds(ut, 128)],
                             ublk_v.at[pl.ds(k * D, D)], sem)
        for k in range(CH):
            pltpu.make_async_copy(uembT_hbm.at[:, pl.ds(0, 128)],
                                  ublk_v.at[pl.ds(k * D, D)], sem).wait()
        for k in range(CH):
            e = c * CH + k
            ulane = jnp.broadcast_to(uvec[k] & 127, (L,))
            mcol = (mvec[k] & 3) * D + iota
            erow = jnp.broadcast_to(e, (L,))
            u_lo = plsc.load_gather(ublk_v, [k * D + iota, ulane])
            u_hi = plsc.load_gather(ublk_v, [k * D + HALF + iota, ulane])
            m_lo = plsc.load_gather(mrows_v, [erow, mcol])
            m_hi = plsc.load_gather(mrows_v, [erow, mcol + HALF])
            p = u_lo * m_lo + u_hi * m_hi
            plsc.store_scatter(prod_v, [iota * BPW + e], p)
        return carry

    lax.fori_loop(0, NCH, chunk_body, 0)

    def group_body(g, carry):
        sl = pl.ds(g * L, L)
        acc = ub_v[sl] + mb_v[sl]
        for j in range(HALF):
            acc = acc + prod_v[pl.ds(j * BPW + g * L, L)]
        o_v[sl] = acc
        return carry

    lax.fori_loop(0, BPW // L, group_body, 0)

    pltpu.sync_copy(o_v, out_hbm.at[pl.ds(base, BPW)])


def kernel(user, mission, user_embedding, mission_embedding, user_bias, mission_bias):
    uembT = user_embedding.T
    membT = mission_embedding.T
    ub = user_bias.reshape(-1)
    mb = mission_bias.reshape(-1)
    out, _ = _mf_sc(user, mission, uembT, membT, ub, mb)
    return out


# speedup vs baseline: 1.1191x; 1.1191x over previous
import functools

import jax
import jax.numpy as jnp
from jax import lax
from jax.experimental import pallas as pl
from jax.experimental.pallas import tpu as pltpu
from jax.experimental.pallas import tpu_sc as plsc

B = 16384
D = 32
L = 16
NUM_CORES = 2
NUM_SUBCORES = 16
NW = NUM_CORES * NUM_SUBCORES
BPW = B // NW
CH = 4
NCH = BPW // CH
HALF = D // 2
NMT = 782
MPK = NMT * 32

_mesh = plsc.VectorSubcoreMesh(core_axis_name="c", subcore_axis_name="s")


@functools.partial(
    pl.kernel,
    mesh=_mesh,
    out_type=(
        jax.ShapeDtypeStruct((B,), jnp.float32),
        jax.ShapeDtypeStruct((NUM_CORES, MPK, 128), jnp.float32),
    ),
    scratch_types=[
        pltpu.VMEM((BPW + L,), jnp.int32),
        pltpu.VMEM((BPW + L,), jnp.int32),
        pltpu.VMEM((BPW,), jnp.int32),
        pltpu.VMEM((CH * D, 128), jnp.float32),
        pltpu.VMEM((2 * D, 128), jnp.float32),
        pltpu.VMEM((2 * D, 128), jnp.float32),
        pltpu.VMEM((BPW, 128), jnp.float32),
        pltpu.VMEM((HALF * BPW,), jnp.float32),
        pltpu.VMEM((BPW,), jnp.float32),
        pltpu.VMEM((BPW,), jnp.float32),
        pltpu.VMEM((BPW,), jnp.float32),
        pltpu.SemaphoreType.DMA,
        pltpu.SemaphoreType.DMA,
        pltpu.SemaphoreType.DMA,
    ],
    compiler_params=pltpu.CompilerParams(
        needs_layout_passes=False,
        disable_bounds_checks=True,
    ),
)
def _mf_sc(user_hbm, mission_hbm, uembT_hbm, membT_hbm, ubias_hbm, mbias_hbm,
           out_hbm, mscr_hbm, uidx_v, midx_v, mpk_v, ublk_v, inblk_v,
           outblk_v, mrows_v, prod_v, ub_v, mb_v, o_v, sem, sem_in, sem_out):
    cid = lax.axis_index("c")
    sid = lax.axis_index("s")
    wid = sid * NUM_CORES + cid
    base = wid * BPW

    pltpu.sync_copy(user_hbm.at[pl.ds(base, BPW)], uidx_v.at[pl.ds(0, BPW)])
    pltpu.sync_copy(mission_hbm.at[pl.ds(base, BPW)], midx_v.at[pl.ds(0, BPW)])
    uidx_v[pl.ds(BPW, L)] = jnp.zeros((L,), jnp.int32)
    midx_v[pl.ds(BPW, L)] = jnp.zeros((L,), jnp.int32)

    cp_ub = pltpu.async_copy(ubias_hbm.at[uidx_v.at[pl.ds(0, BPW)]], ub_v, sem)
    cp_mb = pltpu.async_copy(mbias_hbm.at[midx_v.at[pl.ds(0, BPW)]], mb_v, sem)

    iota = lax.iota(jnp.int32, L)

    TPW = (NMT + NUM_SUBCORES - 1) // NUM_SUBCORES
    t0 = sid * TPW

    def _stage(t, slot):
        off = pl.multiple_of(t * 128, 128)
        pltpu.async_copy(membT_hbm.at[:, pl.ds(off, 128)],
                         inblk_v.at[pl.ds(slot * D, D)], sem_in)

    def _wait_in(slot):
        pltpu.make_async_copy(membT_hbm.at[:, pl.ds(0, 128)],
                              inblk_v.at[pl.ds(slot * D, D)], sem_in).wait()

    def _wait_out(slot):
        pltpu.make_async_copy(outblk_v.at[pl.ds(slot * D, D)],
                              mscr_hbm.at[cid, pl.ds(0, 32), :],
                              sem_out).wait()

    _stage(t0, 0)

    def relayout_body(ti, carry):
        t = t0 + ti
        slot = ti & 1
        valid = t < NMT

        @pl.when(valid & (ti + 1 < TPW) & (t + 1 < NMT))
        def _():
            _stage(t + 1, 1 - slot)

        @pl.when(valid)
        def _():
            _wait_in(slot)

            @pl.when(ti >= 2)
            def _():
                _wait_out(slot)

            def q_body(q, carry2):
                for cg in range(8):
                    d_vec = slot * D + iota + (cg & 1) * L
                    src_lane = jnp.broadcast_to(4 * q + cg // 2, (L,))
                    outblk_v[slot * D + q, pl.ds(cg * L, L)] = plsc.load_gather(
                        inblk_v, [d_vec, src_lane])
                return carry2

            lax.fori_loop(0, D, q_body, 0)
            pltpu.async_copy(outblk_v.at[pl.ds(slot * D, D)],
                             mscr_hbm.at[cid, pl.ds(t * 32, 32), :], sem_out)
        return carry

    lax.fori_loop(0, TPW, relayout_body, 0)
    _wait_out(0)
    _wait_out(1)

    def pack_body(i, carry):
        sl = pl.ds(i * L, L)
        mpk_v[sl] = lax.shift_right_logical(midx_v[sl], 2)
        return carry

    lax.fori_loop(0, BPW // L, pack_body, 0)
    plsc.subcore_barrier()

    cp_m = pltpu.async_copy(mscr_hbm.at[cid].at[mpk_v], mrows_v, sem)
    cp_ub.wait()
    cp_mb.wait()
    cp_m.wait()

    def chunk_body(c, carry):
        uvec = uidx_v[pl.ds(c * CH, L)]
        mvec = midx_v[pl.ds(c * CH, L)]
        for k in range(CH):
            ut = pl.multiple_of((uvec[k] >> 7) * 128, 128)
            pltpu.async_copy(uembT_hbm.at[:, pl.ds(ut, 128)],
                             ublk_v.at[pl.ds(k * D, D)], sem)
        for k in range(CH):
            pltpu.make_async_copy(uembT_hbm.at[:, pl.ds(0, 128)],
                                  ublk_v.at[pl.ds(k * D, D)], sem).wait()
        for k in range(CH):
            e = c * CH + k
            ulane = jnp.broadcast_to(uvec[k] & 127, (L,))
            mcol = (mvec[k] & 3) * D + iota
            erow = jnp.broadcast_to(e, (L,))
            u_lo = plsc.load_gather(ublk_v, [k * D + iota, ulane])
            u_hi = plsc.load_gather(ublk_v, [k * D + HALF + iota, ulane])
            m_lo = plsc.load_gather(mrows_v, [erow, mcol])
            m_hi = plsc.load_gather(mrows_v, [erow, mcol + HALF])
            p = u_lo * m_lo + u_hi * m_hi
            plsc.store_scatter(prod_v, [iota * BPW + e], p)
        return carry

    lax.fori_loop(0, NCH, chunk_body, 0)

    def group_body(g, carry):
        sl = pl.ds(g * L, L)
        acc = ub_v[sl] + mb_v[sl]
        for j in range(HALF):
            acc = acc + prod_v[pl.ds(j * BPW + g * L, L)]
        o_v[sl] = acc
        return carry

    lax.fori_loop(0, BPW // L, group_body, 0)

    pltpu.sync_copy(o_v, out_hbm.at[pl.ds(base, BPW)])


def kernel(user, mission, user_embedding, mission_embedding, user_bias, mission_bias):
    uembT = user_embedding.T
    membT = mission_embedding.T
    ub = user_bias.reshape(-1)
    mb = mission_bias.reshape(-1)
    out, _ = _mf_sc(user, mission, uembT, membT, ub, mb)
    return out


# speedup vs baseline: 1.5668x; 1.4000x over previous
import functools

import jax
import jax.numpy as jnp
from jax import lax
from jax.experimental import pallas as pl
from jax.experimental.pallas import tpu as pltpu
from jax.experimental.pallas import tpu_sc as plsc

B = 16384
D = 32
L = 16
NUM_CORES = 2
NUM_SUBCORES = 16
NW = NUM_CORES * NUM_SUBCORES
BPW = B // NW
CH = 4
NCH = BPW // CH
HALF = D // 2

_mesh = plsc.VectorSubcoreMesh(core_axis_name="c", subcore_axis_name="s")


@functools.partial(
    pl.kernel,
    mesh=_mesh,
    out_type=jax.ShapeDtypeStruct((B,), jnp.float32),
    scratch_types=[
        pltpu.VMEM((BPW + L,), jnp.int32),
        pltpu.VMEM((BPW + L,), jnp.int32),
        pltpu.VMEM((2 * CH * D, 128), jnp.float32),
        pltpu.VMEM((2 * CH * D, 128), jnp.float32),
        pltpu.VMEM((HALF * BPW,), jnp.float32),
        pltpu.VMEM((BPW,), jnp.float32),
        pltpu.VMEM((BPW,), jnp.float32),
        pltpu.VMEM((BPW,), jnp.float32),
        pltpu.SemaphoreType.DMA,
        pltpu.SemaphoreType.DMA,
        pltpu.SemaphoreType.DMA,
    ],
    compiler_params=pltpu.CompilerParams(
        needs_layout_passes=False,
        disable_bounds_checks=True,
    ),
)
def _mf_sc(user_hbm, mission_hbm, uembT_hbm, membT_hbm, ubias_hbm, mbias_hbm,
           out_hbm, uidx_v, midx_v, ublk_v, mblk_v, prod_v, ub_v, mb_v, o_v,
           sem, sem_a, sem_b):
    wid = lax.axis_index("s") * NUM_CORES + lax.axis_index("c")
    base = wid * BPW

    pltpu.sync_copy(user_hbm.at[pl.ds(base, BPW)], uidx_v.at[pl.ds(0, BPW)])
    pltpu.sync_copy(mission_hbm.at[pl.ds(base, BPW)], midx_v.at[pl.ds(0, BPW)])
    uidx_v[pl.ds(BPW, L)] = jnp.zeros((L,), jnp.int32)
    midx_v[pl.ds(BPW, L)] = jnp.zeros((L,), jnp.int32)

    cp_ub = pltpu.async_copy(ubias_hbm.at[uidx_v.at[pl.ds(0, BPW)]], ub_v, sem)
    cp_mb = pltpu.async_copy(mbias_hbm.at[midx_v.at[pl.ds(0, BPW)]], mb_v, sem)
    cp_ub.wait()
    cp_mb.wait()

    iota = lax.iota(jnp.int32, L)

    def _fire(c, slot, semx):
        uvec = uidx_v[pl.ds(c * CH, L)]
        mvec = midx_v[pl.ds(c * CH, L)]
        for k in range(CH):
            ut = pl.multiple_of((uvec[k] >> 7) * 128, 128)
            mt = pl.multiple_of((mvec[k] >> 7) * 128, 128)
            row = (slot * CH + k) * D
            pltpu.async_copy(uembT_hbm.at[:, pl.ds(ut, 128)],
                             ublk_v.at[pl.ds(row, D)], semx)
            pltpu.async_copy(membT_hbm.at[:, pl.ds(mt, 128)],
                             mblk_v.at[pl.ds(row, D)], semx)

    def _drain(slot, semx):
        for k in range(CH):
            row = (slot * CH + k) * D
            pltpu.make_async_copy(uembT_hbm.at[:, pl.ds(0, 128)],
                                  ublk_v.at[pl.ds(row, D)], semx).wait()
            pltpu.make_async_copy(membT_hbm.at[:, pl.ds(0, 128)],
                                  mblk_v.at[pl.ds(row, D)], semx).wait()

    _fire(0, 0, sem_a)

    def chunk_body(c, carry):
        even = (c & 1) == 0
        more = c + 1 < NCH

        @pl.when(jnp.logical_and(even, more))
        def _():
            _fire(c + 1, 1, sem_b)

        @pl.when(jnp.logical_and(jnp.logical_not(even), more))
        def _():
            _fire(c + 1, 0, sem_a)

        @pl.when(even)
        def _():
            _drain(0, sem_a)

        @pl.when(jnp.logical_not(even))
        def _():
            _drain(1, sem_b)

        srow = (c & 1) * CH * D
        uvec = uidx_v[pl.ds(c * CH, L)]
        mvec = midx_v[pl.ds(c * CH, L)]
        for k in range(CH):
            ulane = jnp.broadcast_to(uvec[k] & 127, (L,))
            mlane = jnp.broadcast_to(mvec[k] & 127, (L,))
            u_lo = plsc.load_gather(ublk_v, [srow + k * D + iota, ulane])
            u_hi = plsc.load_gather(ublk_v, [srow + k * D + HALF + iota, ulane])
            m_lo = plsc.load_gather(mblk_v, [srow + k * D + iota, mlane])
            m_hi = plsc.load_gather(mblk_v, [srow + k * D + HALF + iota, mlane])
            p = u_lo * m_lo + u_hi * m_hi
            plsc.store_scatter(prod_v, [iota * BPW + (c * CH + k)], p)
        return carry

    lax.fori_loop(0, NCH, chunk_body, 0)

    def group_body(g, carry):
        sl = pl.ds(g * L, L)
        acc = ub_v[sl] + mb_v[sl]
        for j in range(HALF):
            acc = acc + prod_v[pl.ds(j * BPW + g * L, L)]
        o_v[sl] = acc
        return carry

    lax.fori_loop(0, BPW // L, group_body, 0)

    pltpu.sync_copy(o_v, out_hbm.at[pl.ds(base, BPW)])


def kernel(user, mission, user_embedding, mission_embedding, user_bias, mission_bias):
    uembT = user_embedding.T
    membT = mission_embedding.T
    ub = user_bias.reshape(-1)
    mb = mission_bias.reshape(-1)
    return _mf_sc(user, mission, uembT, membT, ub, mb)
